# baseline (speedup 1.0000x reference)
import functools

import jax
import jax.numpy as jnp
from jax import lax
from jax.experimental import pallas as pl
from jax.experimental.pallas import tpu as pltpu
from jax.experimental.pallas import tpu_sc as plsc

F32 = jnp.float32
I32 = jnp.int32

N_NODES = 50000
HEADS = 16
C_CHUNK = 2560
K_CHUNKS = 20
NP = C_CHUNK * K_CHUNKS
NP8 = NP + 8
CP_ROWS = 2688
ZROWS = 168
DRAIN = C_CHUNK // 16

E_RAW = 800000
E_TOT = E_RAW + N_NODES
EB = 1024
NBLK = 52
E16 = EB * NBLK
E_PAD = E16 * 16
M_BATCH = 64
QCAP = 3136
QTRIG = 1024

BN = 512


def _iota16():
  return lax.iota(I32, 16)


def _proj(x, w, ss, sd):
  npad, cinp = x.shape
  d = w.shape[1]

  def body(x_ref, w_ref, ss_ref, sd_ref, h_ref, als_ref, ald_ref):
    h = jnp.dot(x_ref[...], w_ref[...], preferred_element_type=F32)
    h_ref[...] = h
    als_ref[...] = jnp.dot(h, ss_ref[...], preferred_element_type=F32)
    ald_ref[...] = jnp.dot(h, sd_ref[...], preferred_element_type=F32)

  return pl.pallas_call(
      body,
      grid=(npad // BN,),
      in_specs=[
          pl.BlockSpec((BN, cinp), lambda i: (i, 0)),
          pl.BlockSpec((cinp, d), lambda i: (0, 0)),
          pl.BlockSpec((d, HEADS), lambda i: (0, 0)),
          pl.BlockSpec((d, HEADS), lambda i: (0, 0)),
      ],
      out_specs=[
          pl.BlockSpec((BN, d), lambda i: (i, 0)),
          pl.BlockSpec((BN, HEADS), lambda i: (i, 0)),
          pl.BlockSpec((BN, HEADS), lambda i: (i, 0)),
      ],
      out_shape=[
          jax.ShapeDtypeStruct((npad, d), F32),
          jax.ShapeDtypeStruct((npad, HEADS), F32),
          jax.ShapeDtypeStruct((npad, HEADS), F32),
      ],
  )(x, w, ss, sd)


def _finalize(accraw, em, bias, relu):
  npad, r = accraw.shape
  d = r - HEADS

  def body(a_ref, e_ref, b_ref, o_ref):
    un = a_ref[:, :d]
    den = a_ref[:, d:]
    rec = 1.0 / (den + 1e-16)
    rexp = jnp.dot(rec, e_ref[...], preferred_element_type=F32)
    o = un * rexp + b_ref[...]
    if relu:
      o = jnp.maximum(o, 0.0)
    o_ref[...] = o

  return pl.pallas_call(
      body,
      grid=(npad // BN,),
      in_specs=[
          pl.BlockSpec((BN, r), lambda i: (i, 0)),
          pl.BlockSpec((HEADS, d), lambda i: (0, 0)),
          pl.BlockSpec((1, d), lambda i: (0, 0)),
      ],
      out_specs=pl.BlockSpec((BN, d), lambda i: (i, 0)),
      out_shape=jax.ShapeDtypeStruct((npad, d), F32),
  )(accraw, em, bias)


def _mlp(x, fcw, fcb, ow, ob):
  npad, din = x.shape
  dmid = fcw.shape[1]
  dout = ow.shape[1]

  def body(x_ref, fw_ref, fb_ref, ow_ref, ob_ref, o_ref):
    t = jnp.dot(x_ref[...], fw_ref[...], preferred_element_type=F32)
    t = t + fb_ref[...]
    o_ref[...] = jnp.dot(t, ow_ref[...], preferred_element_type=F32) + ob_ref[...]

  return pl.pallas_call(
      body,
      grid=(npad // BN,),
      in_specs=[
          pl.BlockSpec((BN, din), lambda i: (i, 0)),
          pl.BlockSpec((din, dmid), lambda i: (0, 0)),
          pl.BlockSpec((1, dmid), lambda i: (0, 0)),
          pl.BlockSpec((dmid, dout), lambda i: (0, 0)),
          pl.BlockSpec((1, dout), lambda i: (0, 0)),
      ],
      out_specs=pl.BlockSpec((BN, dout), lambda i: (i, 0)),
      out_shape=jax.ShapeDtypeStruct((npad, dout), F32),
  )(x, fcw, fcb, ow, ob)


def _sc_edge(zeros, src, dst, als, ald8, h, mhat, d):
  r = d + HEADS
  cph = d // HEADS
  mesh = plsc.VectorSubcoreMesh(core_axis_name="c", subcore_axis_name="s")
  nvec = M_BATCH // 16

  def body(z_hbm, src_hbm, dst_hbm, als_hbm, ald_hbm, h_hbm, mh_hbm, out_hbm,
           acc, eblkd, eblks, qsrc, qdst, bldst, sldst, als_v, ald_v, h_v,
           contrib, mh_v, esems, gsems, ssems):
    cid = lax.axis_index("c")
    sid = lax.axis_index("s")
    pltpu.sync_copy(mh_hbm, mh_v)

    def fire_blk(bi, sl):
      base = sid * E16 + bi * EB
      pltpu.make_async_copy(dst_hbm.at[pl.ds(base, EB)], eblkd.at[sl],
                            esems.at[2 * sl]).start()
      pltpu.make_async_copy(src_hbm.at[pl.ds(base, EB)], eblks.at[sl],
                            esems.at[2 * sl + 1]).start()

    def wait_blk(bi, sl):
      base = sid * E16 + bi * EB
      pltpu.make_async_copy(dst_hbm.at[pl.ds(base, EB)], eblkd.at[sl],
                            esems.at[2 * sl]).wait()
      pltpu.make_async_copy(src_hbm.at[pl.ds(base, EB)], eblks.at[sl],
                            esems.at[2 * sl + 1]).wait()

    def fire_batch(b, sl, lo):
      for k in range(nvec):
        bldst[sl, pl.ds(16 * k, 16)] = (
            qdst[pl.ds(b * M_BATCH + 16 * k, 16)] - lo)
      idx = qsrc.at[pl.ds(b * M_BATCH, M_BATCH)]
      pltpu.make_async_copy(als_hbm.at[idx], als_v.at[sl],
                            gsems.at[3 * sl]).start()
      pltpu.make_async_copy(h_hbm.at[idx], h_v.at[sl],
                            gsems.at[3 * sl + 1]).start()
      idxd = qdst.at[pl.ds(b * M_BATCH, M_BATCH)]
      pltpu.make_async_copy(ald_hbm.at[idxd], ald_v.at[sl],
                            gsems.at[3 * sl + 2]).start()

    def wait_scatter(sl):
      pltpu.make_async_copy(contrib.at[sl], acc.at[sldst.at[sl]],
                            ssems.at[sl]).wait()

    def proc_batch(b, sl):
      idx = qsrc.at[pl.ds(b * M_BATCH, M_BATCH)]
      idxd = qdst.at[pl.ds(b * M_BATCH, M_BATCH)]
      pltpu.make_async_copy(als_hbm.at[idx], als_v.at[sl],
                            gsems.at[3 * sl]).wait()
      pltpu.make_async_copy(h_hbm.at[idx], h_v.at[sl],
                            gsems.at[3 * sl + 1]).wait()
      pltpu.make_async_copy(ald_hbm.at[idxd], ald_v.at[sl],
                            gsems.at[3 * sl + 2]).wait()

      @pl.when(b >= 2)
      def _():
        wait_scatter(sl)

      for k in range(M_BATCH // 16):
        sldst[sl, pl.ds(16 * k, 16)] = bldst[sl, pl.ds(16 * k, 16)]

      mh = mh_v[...]

      @plsc.parallel_loop(0, M_BATCH, unroll=4)
      def _(i):
        a = als_v[sl, i, :]
        ad = ald_v[sl, i, :]
        e = a + ad
        e = jnp.where(e > 0.0, e, 0.2 * e)
        p = jnp.exp(e - mh)
        contrib[sl, i, pl.ds(d, 16)] = p
        for k in range(d // 16):
          hk = h_v[sl, i, pl.ds(16 * k, 16)]
          if cph == 16:
            pk = p[k]
          else:
            pk = jnp.where(_iota16() < 8, p[2 * k], p[2 * k + 1])
          contrib[sl, i, pl.ds(16 * k, 16)] = pk * hk

      pltpu.async_copy(contrib.at[sl], acc.at[sldst.at[sl]],
                       ssems.at[sl], add=True)

    def run_queue(nbt, lo):
      @pl.when(nbt > 0)
      def _():
        fire_batch(0, 0, lo)

      def pair_body(pr, carry):
        b0 = 2 * pr
        b1 = b0 + 1

        @pl.when(b1 < nbt)
        def _():
          fire_batch(b1, 1, lo)

        proc_batch(b0, 0)

        @pl.when(b0 + 2 < nbt)
        def _():
          fire_batch(b0 + 2, 0, lo)

        @pl.when(b1 < nbt)
        def _():
          proc_batch(b1, 1)

        return carry

      lax.fori_loop(0, (nbt + 1) // 2, pair_body, jnp.asarray(0, I32))

      @pl.when(nbt >= 1)
      def _():
        wait_scatter(0)

      @pl.when(nbt >= 2)
      def _():
        wait_scatter(1)

    def filter_block(sl, qcnt, lo, hi):
      @plsc.parallel_loop(0, EB // 16, unroll=4, carry=qcnt)
      def vec_body(v, qcnt):
        d16 = eblkd[sl, pl.ds(v * 16, 16)]
        s16 = eblks[sl, pl.ds(v * 16, 16)]
        m = (d16 >= lo) & (d16 < hi)
        nm = jnp.sum(jnp.where(m, 1, 0))
        plsc.store_compressed(qsrc.at[pl.ds(qcnt, 16)], s16, mask=m)
        plsc.store_compressed(qdst.at[pl.ds(qcnt, 16)], d16, mask=m)
        return qcnt + nm

      return vec_body

    @pl.loop(0, K_CHUNKS // 2)
    def _(pp):
      chunk = pp * 2 + cid
      lo = chunk * C_CHUNK
      hi = lo + C_CHUNK
      pltpu.sync_copy(z_hbm.at[pl.ds(sid * ZROWS, ZROWS)],
                      acc.at[pl.ds(sid * ZROWS, ZROWS)])
      plsc.subcore_barrier()

      fire_blk(0, 0)

      def pair_blk(j, qcnt):
        b0 = 2 * j
        fire_blk(b0 + 1, 1)
        wait_blk(b0, 0)
        qcnt = filter_block(0, qcnt, lo, hi)

        @pl.when(b0 + 2 < NBLK)
        def _():
          fire_blk(b0 + 2, 0)

        wait_blk(b0 + 1, 1)
        qcnt = filter_block(1, qcnt, lo, hi)

        def drain(qcnt):
          nb = qcnt // M_BATCH
          run_queue(nb, lo)
          rem = qcnt - nb * M_BATCH
          for k in range(nvec):
            qsrc[pl.ds(16 * k, 16)] = qsrc[pl.ds(nb * M_BATCH + 16 * k, 16)]
            qdst[pl.ds(16 * k, 16)] = qdst[pl.ds(nb * M_BATCH + 16 * k, 16)]
          return rem

        return lax.cond(qcnt >= QTRIG, drain, lambda q: q, qcnt)

      qcnt = lax.fori_loop(0, NBLK // 2, pair_blk, jnp.asarray(0, I32))

      for k in range(nvec):
        qsrc[pl.ds(qcnt + 16 * k, 16)] = jnp.zeros((16,), I32)
        qdst[pl.ds(qcnt + 16 * k, 16)] = jnp.full((16,), 1, I32) * hi
      run_queue((qcnt + M_BATCH - 1) // M_BATCH, lo)

      plsc.subcore_barrier()
      pltpu.sync_copy(acc.at[pl.ds(sid * DRAIN, DRAIN)],
                      out_hbm.at[pl.ds(lo + sid * DRAIN, DRAIN)])
      plsc.subcore_barrier()

  kernel = pl.kernel(
      body,
      out_type=jax.ShapeDtypeStruct((NP, r), F32),
      mesh=mesh,
      compiler_params=pltpu.CompilerParams(needs_layout_passes=False,
                                           use_tc_tiling_on_sc=False),
      scratch_types=[
          pltpu.VMEM_SHARED((CP_ROWS, r), F32),
          pltpu.VMEM((2, EB), I32),
          pltpu.VMEM((2, EB), I32),
          pltpu.VMEM((QCAP,), I32),
          pltpu.VMEM((QCAP,), I32),
          pltpu.VMEM((2, M_BATCH), I32),
          pltpu.VMEM((2, M_BATCH), I32),
          pltpu.VMEM((2, M_BATCH, HEADS), F32),
          pltpu.VMEM((2, M_BATCH, HEADS), F32),
          pltpu.VMEM((2, M_BATCH, d), F32),
          pltpu.VMEM((2, M_BATCH, r), F32),
          pltpu.VMEM((HEADS,), F32),
          pltpu.SemaphoreType.DMA((4,)),
          pltpu.SemaphoreType.DMA((6,)),
          pltpu.SemaphoreType.DMA((2,)),
      ],
  )
  return kernel(zeros, src, dst, als, ald8, h, mhat)


LAYER_CFG = [(16, 256, 16), (256, 256, 16), (256, 256, 16), (256, 128, 8)]


def _leaky(x):
  return jnp.where(x > 0.0, x, 0.2 * x)


def _block_diag(a):
  heads, cph = a.shape
  eye = jnp.eye(heads, dtype=a.dtype)
  return (a[:, :, None] * eye[:, None, :]).reshape(heads * cph, heads)


def _expand_mat(d):
  cph = d // HEADS
  cols = jnp.arange(d) // cph
  return (cols[None, :] == jnp.arange(HEADS)[:, None]).astype(F32)


def kernel(x, edge_index, W0, asrc0, adst0, b0, W1, asrc1, adst1, b1,
           W2, asrc2, adst2, b2, W3, asrc3, adst3, b3,
           fc_W, fc_b, out_W, out_b):
  n = x.shape[0]
  loop = jnp.arange(n, dtype=edge_index.dtype)
  src = jnp.concatenate([edge_index[0], loop])
  dst = jnp.concatenate([edge_index[1], loop])
  src = jnp.concatenate([src, jnp.zeros((E_PAD - E_TOT,), I32)])
  dst = jnp.concatenate([dst, jnp.full((E_PAD - E_TOT,), -1, I32)])

  xin = jnp.pad(x, ((0, NP - n), (0, 16 - x.shape[1])))
  params = [(W0, asrc0, adst0, b0), (W1, asrc1, adst1, b1),
            (W2, asrc2, adst2, b2), (W3, asrc3, adst3, b3)]

  for li, (w, a_s, a_d, b) in enumerate(params):
    cinp, d, cph = LAYER_CFG[li]
    wpad = jnp.pad(w, ((0, cinp - w.shape[0]), (0, 0)))
    ss = _block_diag(a_s)
    sd = _block_diag(a_d)
    h, als, ald = _proj(xin, wpad, ss, sd)
    mhat = _leaky(jnp.max(als, axis=0) + jnp.max(ald, axis=0))
    ald8 = jnp.pad(ald, ((0, 8), (0, 0)))
    zeros = jnp.zeros((CP_ROWS, d + HEADS), F32)
    accraw = _sc_edge(zeros, src, dst, als, ald8, h, mhat, d)
    xin = _finalize(accraw, _expand_mat(d), b.reshape(1, d), relu=(li < 3))

  y = _mlp(xin, fc_W, fc_b.reshape(1, -1), out_W, out_b.reshape(1, -1))
  return y[:n]


# speedup vs baseline: 1.0297x; 1.0297x over previous
import functools

import jax
import jax.numpy as jnp
from jax import lax
from jax.experimental import pallas as pl
from jax.experimental.pallas import tpu as pltpu
from jax.experimental.pallas import tpu_sc as plsc

F32 = jnp.float32
I32 = jnp.int32

N_NODES = 50000
HEADS = 16
C_CHUNK = 2560
K_CHUNKS = 20
NP = C_CHUNK * K_CHUNKS
NP8 = NP + 8
CP_ROWS = 2688
ZROWS = 168
DRAIN = C_CHUNK // 16

E_RAW = 800000
E_TOT = E_RAW + N_NODES
EB = 1024
NBLK = 52
E16 = EB * NBLK
E_PAD = E16 * 16
M_BATCH = 64
QCAP = 3136
QTRIG = 1024

BN = 512


def _iota16():
  return lax.iota(I32, 16)


def _proj(x, w, ss, sd):
  npad, cinp = x.shape
  d = w.shape[1]

  def body(x_ref, w_ref, ss_ref, sd_ref, h_ref, als_ref, ald_ref):
    h = jnp.dot(x_ref[...], w_ref[...], preferred_element_type=F32)
    h_ref[...] = h
    als_ref[...] = jnp.dot(h, ss_ref[...], preferred_element_type=F32)
    ald_ref[...] = jnp.dot(h, sd_ref[...], preferred_element_type=F32)

  return pl.pallas_call(
      body,
      grid=(npad // BN,),
      in_specs=[
          pl.BlockSpec((BN, cinp), lambda i: (i, 0)),
          pl.BlockSpec((cinp, d), lambda i: (0, 0)),
          pl.BlockSpec((d, HEADS), lambda i: (0, 0)),
          pl.BlockSpec((d, HEADS), lambda i: (0, 0)),
      ],
      out_specs=[
          pl.BlockSpec((BN, d), lambda i: (i, 0)),
          pl.BlockSpec((BN, HEADS), lambda i: (i, 0)),
          pl.BlockSpec((BN, HEADS), lambda i: (i, 0)),
      ],
      out_shape=[
          jax.ShapeDtypeStruct((npad, d), F32),
          jax.ShapeDtypeStruct((npad, HEADS), F32),
          jax.ShapeDtypeStruct((npad, HEADS), F32),
      ],
  )(x, w, ss, sd)


def _finalize(accraw, em, bias, relu):
  npad, r = accraw.shape
  d = r - HEADS

  def body(a_ref, e_ref, b_ref, o_ref):
    un = a_ref[:, :d]
    den = a_ref[:, d:]
    rec = 1.0 / (den + 1e-16)
    rexp = jnp.dot(rec, e_ref[...], preferred_element_type=F32)
    o = un * rexp + b_ref[...]
    if relu:
      o = jnp.maximum(o, 0.0)
    o_ref[...] = o

  return pl.pallas_call(
      body,
      grid=(npad // BN,),
      in_specs=[
          pl.BlockSpec((BN, r), lambda i: (i, 0)),
          pl.BlockSpec((HEADS, d), lambda i: (0, 0)),
          pl.BlockSpec((1, d), lambda i: (0, 0)),
      ],
      out_specs=pl.BlockSpec((BN, d), lambda i: (i, 0)),
      out_shape=jax.ShapeDtypeStruct((npad, d), F32),
  )(accraw, em, bias)


def _mlp(x, fcw, fcb, ow, ob):
  npad, din = x.shape
  dmid = fcw.shape[1]
  dout = ow.shape[1]

  def body(x_ref, fw_ref, fb_ref, ow_ref, ob_ref, o_ref):
    t = jnp.dot(x_ref[...], fw_ref[...], preferred_element_type=F32)
    t = t + fb_ref[...]
    o_ref[...] = jnp.dot(t, ow_ref[...], preferred_element_type=F32) + ob_ref[...]

  return pl.pallas_call(
      body,
      grid=(npad // BN,),
      in_specs=[
          pl.BlockSpec((BN, din), lambda i: (i, 0)),
          pl.BlockSpec((din, dmid), lambda i: (0, 0)),
          pl.BlockSpec((1, dmid), lambda i: (0, 0)),
          pl.BlockSpec((dmid, dout), lambda i: (0, 0)),
          pl.BlockSpec((1, dout), lambda i: (0, 0)),
      ],
      out_specs=pl.BlockSpec((BN, dout), lambda i: (i, 0)),
      out_shape=jax.ShapeDtypeStruct((npad, dout), F32),
  )(x, fcw, fcb, ow, ob)


def _sc_edge(zeros, src, dst, als, ald8, h, mhat, d):
  r = d + HEADS
  cph = d // HEADS
  mesh = plsc.VectorSubcoreMesh(core_axis_name="c", subcore_axis_name="s")
  nvec = M_BATCH // 16

  def body(z_hbm, src_hbm, dst_hbm, als_hbm, ald_hbm, h_hbm, mh_hbm, out_hbm,
           acc, eblkd, eblks, qsrc, qdst, bldst, sldst, als_v, ald_v, h_v,
           contrib, mh_v, esems, gsems, ssems):
    cid = lax.axis_index("c")
    sid = lax.axis_index("s")
    pltpu.sync_copy(mh_hbm, mh_v)

    def fire_blk(bi, sl):
      base = sid * E16 + bi * EB
      pltpu.make_async_copy(dst_hbm.at[pl.ds(base, EB)], eblkd.at[sl],
                            esems.at[2 * sl]).start()
      pltpu.make_async_copy(src_hbm.at[pl.ds(base, EB)], eblks.at[sl],
                            esems.at[2 * sl + 1]).start()

    def wait_blk(bi, sl):
      base = sid * E16 + bi * EB
      pltpu.make_async_copy(dst_hbm.at[pl.ds(base, EB)], eblkd.at[sl],
                            esems.at[2 * sl]).wait()
      pltpu.make_async_copy(src_hbm.at[pl.ds(base, EB)], eblks.at[sl],
                            esems.at[2 * sl + 1]).wait()

    def fire_batch(b, sl, lo):
      for k in range(nvec):
        bldst[sl, pl.ds(16 * k, 16)] = (
            qdst[pl.ds(b * M_BATCH + 16 * k, 16)] - lo)
      idx = qsrc.at[pl.ds(b * M_BATCH, M_BATCH)]
      pltpu.make_async_copy(als_hbm.at[idx], als_v.at[sl],
                            gsems.at[3 * sl]).start()
      pltpu.make_async_copy(h_hbm.at[idx], h_v.at[sl],
                            gsems.at[3 * sl + 1]).start()
      idxd = qdst.at[pl.ds(b * M_BATCH, M_BATCH)]
      pltpu.make_async_copy(ald_hbm.at[idxd], ald_v.at[sl],
                            gsems.at[3 * sl + 2]).start()

    def wait_scatter(sl):
      pltpu.make_async_copy(contrib.at[sl], acc.at[sldst.at[sl]],
                            ssems.at[sl]).wait()

    def proc_batch(b, sl):
      idx = qsrc.at[pl.ds(b * M_BATCH, M_BATCH)]
      idxd = qdst.at[pl.ds(b * M_BATCH, M_BATCH)]
      pltpu.make_async_copy(als_hbm.at[idx], als_v.at[sl],
                            gsems.at[3 * sl]).wait()
      pltpu.make_async_copy(h_hbm.at[idx], h_v.at[sl],
                            gsems.at[3 * sl + 1]).wait()
      pltpu.make_async_copy(ald_hbm.at[idxd], ald_v.at[sl],
                            gsems.at[3 * sl + 2]).wait()

      @pl.when(b >= 2)
      def _():
        wait_scatter(sl)

      for k in range(M_BATCH // 16):
        sldst[sl, pl.ds(16 * k, 16)] = bldst[sl, pl.ds(16 * k, 16)]

      mh = mh_v[...]

      @plsc.parallel_loop(0, M_BATCH, unroll=8)
      def _(i):
        a = als_v[sl, i, :]
        ad = ald_v[sl, i, :]
        e = a + ad
        e = jnp.where(e > 0.0, e, 0.2 * e)
        p = jnp.exp(e - mh)
        contrib[sl, i, pl.ds(d, 16)] = p
        for k in range(d // 16):
          hk = h_v[sl, i, pl.ds(16 * k, 16)]
          if cph == 16:
            pk = p[k]
          else:
            pk = jnp.where(_iota16() < 8, p[2 * k], p[2 * k + 1])
          contrib[sl, i, pl.ds(16 * k, 16)] = pk * hk

      pltpu.async_copy(contrib.at[sl], acc.at[sldst.at[sl]],
                       ssems.at[sl], add=True)

    def run_queue(nbt, lo):
      @pl.when(nbt > 0)
      def _():
        fire_batch(0, 0, lo)

      def pair_body(pr, carry):
        b0 = 2 * pr
        b1 = b0 + 1

        @pl.when(b1 < nbt)
        def _():
          fire_batch(b1, 1, lo)

        proc_batch(b0, 0)

        @pl.when(b0 + 2 < nbt)
        def _():
          fire_batch(b0 + 2, 0, lo)

        @pl.when(b1 < nbt)
        def _():
          proc_batch(b1, 1)

        return carry

      lax.fori_loop(0, (nbt + 1) // 2, pair_body, jnp.asarray(0, I32))

      @pl.when(nbt >= 1)
      def _():
        wait_scatter(0)

      @pl.when(nbt >= 2)
      def _():
        wait_scatter(1)

    def filter_block(sl, qcnt, lo, hi):
      @plsc.parallel_loop(0, EB // 16, unroll=4, carry=qcnt)
      def vec_body(v, qcnt):
        d16 = eblkd[sl, pl.ds(v * 16, 16)]
        s16 = eblks[sl, pl.ds(v * 16, 16)]
        m = (d16 >= lo) & (d16 < hi)
        nm = jnp.sum(jnp.where(m, 1, 0))
        plsc.store_compressed(qsrc.at[pl.ds(qcnt, 16)], s16, mask=m)
        plsc.store_compressed(qdst.at[pl.ds(qcnt, 16)], d16, mask=m)
        return qcnt + nm

      return vec_body

    @pl.loop(0, K_CHUNKS // 2)
    def _(pp):
      chunk = pp * 2 + cid
      lo = chunk * C_CHUNK
      hi = lo + C_CHUNK
      pltpu.sync_copy(z_hbm.at[pl.ds(sid * ZROWS, ZROWS)],
                      acc.at[pl.ds(sid * ZROWS, ZROWS)])
      plsc.subcore_barrier()

      fire_blk(0, 0)

      def pair_blk(j, qcnt):
        b0 = 2 * j
        fire_blk(b0 + 1, 1)
        wait_blk(b0, 0)
        qcnt = filter_block(0, qcnt, lo, hi)

        @pl.when(b0 + 2 < NBLK)
        def _():
          fire_blk(b0 + 2, 0)

        wait_blk(b0 + 1, 1)
        qcnt = filter_block(1, qcnt, lo, hi)

        def drain(qcnt):
          nb = qcnt // M_BATCH
          run_queue(nb, lo)
          rem = qcnt - nb * M_BATCH
          for k in range(nvec):
            qsrc[pl.ds(16 * k, 16)] = qsrc[pl.ds(nb * M_BATCH + 16 * k, 16)]
            qdst[pl.ds(16 * k, 16)] = qdst[pl.ds(nb * M_BATCH + 16 * k, 16)]
          return rem

        return lax.cond(qcnt >= QTRIG, drain, lambda q: q, qcnt)

      qcnt = lax.fori_loop(0, NBLK // 2, pair_blk, jnp.asarray(0, I32))

      for k in range(nvec):
        qsrc[pl.ds(qcnt + 16 * k, 16)] = jnp.zeros((16,), I32)
        qdst[pl.ds(qcnt + 16 * k, 16)] = jnp.full((16,), 1, I32) * hi
      run_queue((qcnt + M_BATCH - 1) // M_BATCH, lo)

      plsc.subcore_barrier()
      pltpu.sync_copy(acc.at[pl.ds(sid * DRAIN, DRAIN)],
                      out_hbm.at[pl.ds(lo + sid * DRAIN, DRAIN)])
      plsc.subcore_barrier()

  kernel = pl.kernel(
      body,
      out_type=jax.ShapeDtypeStruct((NP, r), F32),
      mesh=mesh,
      compiler_params=pltpu.CompilerParams(needs_layout_passes=False,
                                           use_tc_tiling_on_sc=False),
      scratch_types=[
          pltpu.VMEM_SHARED((CP_ROWS, r), F32),
          pltpu.VMEM((2, EB), I32),
          pltpu.VMEM((2, EB), I32),
          pltpu.VMEM((QCAP,), I32),
          pltpu.VMEM((QCAP,), I32),
          pltpu.VMEM((2, M_BATCH), I32),
          pltpu.VMEM((2, M_BATCH), I32),
          pltpu.VMEM((2, M_BATCH, HEADS), F32),
          pltpu.VMEM((2, M_BATCH, HEADS), F32),
          pltpu.VMEM((2, M_BATCH, d), F32),
          pltpu.VMEM((2, M_BATCH, r), F32),
          pltpu.VMEM((HEADS,), F32),
          pltpu.SemaphoreType.DMA((4,)),
          pltpu.SemaphoreType.DMA((6,)),
          pltpu.SemaphoreType.DMA((2,)),
      ],
  )
  return kernel(zeros, src, dst, als, ald8, h, mhat)


LAYER_CFG = [(16, 256, 16), (256, 256, 16), (256, 256, 16), (256, 128, 8)]


def _leaky(x):
  return jnp.where(x > 0.0, x, 0.2 * x)


def _block_diag(a):
  heads, cph = a.shape
  eye = jnp.eye(heads, dtype=a.dtype)
  return (a[:, :, None] * eye[:, None, :]).reshape(heads * cph, heads)


def _expand_mat(d):
  cph = d // HEADS
  cols = jnp.arange(d) // cph
  return (cols[None, :] == jnp.arange(HEADS)[:, None]).astype(F32)


def kernel(x, edge_index, W0, asrc0, adst0, b0, W1, asrc1, adst1, b1,
           W2, asrc2, adst2, b2, W3, asrc3, adst3, b3,
           fc_W, fc_b, out_W, out_b):
  n = x.shape[0]
  loop = jnp.arange(n, dtype=edge_index.dtype)
  src = jnp.concatenate([edge_index[0], loop])
  dst = jnp.concatenate([edge_index[1], loop])
  src = jnp.concatenate([src, jnp.zeros((E_PAD - E_TOT,), I32)])
  dst = jnp.concatenate([dst, jnp.full((E_PAD - E_TOT,), -1, I32)])

  xin = jnp.pad(x, ((0, NP - n), (0, 16 - x.shape[1])))
  params = [(W0, asrc0, adst0, b0), (W1, asrc1, adst1, b1),
            (W2, asrc2, adst2, b2), (W3, asrc3, adst3, b3)]

  for li, (w, a_s, a_d, b) in enumerate(params):
    cinp, d, cph = LAYER_CFG[li]
    wpad = jnp.pad(w, ((0, cinp - w.shape[0]), (0, 0)))
    ss = _block_diag(a_s)
    sd = _block_diag(a_d)
    h, als, ald = _proj(xin, wpad, ss, sd)
    mhat = _leaky(jnp.max(als, axis=0) + jnp.max(ald, axis=0))
    ald8 = jnp.pad(ald, ((0, 8), (0, 0)))
    zeros = jnp.zeros((CP_ROWS, d + HEADS), F32)
    accraw = _sc_edge(zeros, src, dst, als, ald8, h, mhat, d)
    xin = _finalize(accraw, _expand_mat(d), b.reshape(1, d), relu=(li < 3))

  y = _mlp(xin, fc_W, fc_b.reshape(1, -1), out_W, out_b.reshape(1, -1))
  return y[:n]
